# Initial kernel scaffold; baseline (speedup 1.0000x reference)
import jax
import jax.numpy as jnp
from jax.experimental import pallas as pl


def kernel(modality_ids, table):
    raise NotImplementedError("write your pallas kernel here")


# speedup vs baseline: 4.0667x; 4.0667x over previous
import jax
import jax.numpy as jnp
from jax.experimental import pallas as pl
from jax.experimental.pallas import tpu as pltpu

_NUM_MODALITIES = 3
_HIDDEN = 2048
_BLK = 512


def _embed_body(ids_ref, table_ref, out_ref):
    ids = ids_ref[0, 0, :].reshape(_BLK, 1)
    r0 = table_ref[0:1, :]
    r1 = table_ref[1:2, :]
    r2 = table_ref[2:3, :]
    out_ref[...] = jnp.where(ids == 0, r0, jnp.where(ids == 1, r1, r2))


def kernel(modality_ids, table):
    b, s = modality_ids.shape
    n = b * s
    nblk = n // _BLK
    ids3 = modality_ids.reshape(nblk, 1, _BLK).astype(jnp.int32)

    out = pl.pallas_call(
        _embed_body,
        grid=(nblk,),
        in_specs=[
            pl.BlockSpec((1, 1, _BLK), lambda i: (i, 0, 0)),
            pl.BlockSpec((_NUM_MODALITIES, _HIDDEN), lambda i: (0, 0)),
        ],
        out_specs=pl.BlockSpec((_BLK, _HIDDEN), lambda i: (i, 0)),
        out_shape=jax.ShapeDtypeStruct((n, _HIDDEN), jnp.float32),
    )(ids3, table)
    return out.reshape(b, s, _HIDDEN)
